# baseline (speedup 1.0000x reference)
import functools

import jax
import jax.numpy as jnp
from jax import lax
from jax.experimental import pallas as pl
from jax.experimental.pallas import tpu as pltpu
from jax.experimental.pallas import tpu_sc as plsc

_NC = 2
_NS = 16
_NW = _NC * _NS

_VOCAB = 1000000
_D = 64
_SEQ = 200
_BATCH = 4096
_CB = 128
_NCH = _VOCAB // _CB
_CH_MAIN = (_NCH // _NW) * _NW


def _detile_chunk(in_b, out_b, iotas, n_tok):
    ngrp = n_tok // 16
    fu = 4

    def fbody(f4, carry):
        f0 = f4 * fu
        fsplats = [jnp.full((16,), f0 + d, jnp.int32) for d in range(fu)]
        vals = [in_b[f0 + d, pl.ds(j * 16, 16)]
                for d in range(fu) for j in range(ngrp)]
        for d in range(fu):
            for j in range(ngrp):
                plsc.store_scatter(out_b, [iotas[j], fsplats[d]],
                                   vals[d * ngrp + j])
        return carry

    lax.fori_loop(0, _D // fu, fbody, 0)


def _relayout_kernel(tt, t2, in_v, out_v, tail_v, isem0, isem1, osem0, osem1):
    w = lax.axis_index("s") * _NC + lax.axis_index("c")
    isems = (isem0, isem1)
    osems = (osem0, osem1)
    iota = lax.iota(jnp.int32, 16)
    iotas = [iota + j * 16 for j in range(_CB // 16)]

    def in_slice(c):
        return tt.at[:, pl.ds(c * _CB, _CB)]

    def out_slice(c):
        return t2.at[pl.ds(c * _CB, _CB)]

    pltpu.async_copy(in_slice(w), in_v.at[0], isems[0])

    def outer(kk, carry):
        for b in range(2):
            j = 2 * kk + b
            c = w + j * _NW
            pltpu.make_async_copy(in_slice(c), in_v.at[b], isems[b]).wait()

            @pl.when(c + _NW < _CH_MAIN)
            def _():
                pltpu.async_copy(in_slice(c + _NW), in_v.at[1 - b], isems[1 - b])

            @pl.when(j >= 2)
            def _():
                pltpu.make_async_copy(out_v.at[b], out_slice(c), osems[b]).wait()

            _detile_chunk(in_v.at[b], out_v.at[b], iotas, _CB)
            pltpu.async_copy(out_v.at[b], out_slice(c), osems[b])
        return carry

    lax.fori_loop(0, _CH_MAIN // _NW // 2, outer, 0)
    for b in range(2):
        pltpu.make_async_copy(out_v.at[b], out_slice(0), osems[b]).wait()

    c_tail = w + _CH_MAIN

    @pl.when(c_tail < _NCH)
    def _():
        pltpu.sync_copy(in_slice(c_tail), in_v.at[0])
        _detile_chunk(in_v.at[0], out_v.at[0], iotas, _CB)
        pltpu.sync_copy(out_v.at[0], out_slice(c_tail))

    @pl.when(c_tail == _NCH)
    def _():
        n_tail = _VOCAB - _NCH * _CB
        pltpu.sync_copy(tt.at[:, pl.ds(_NCH * _CB, n_tail)], tail_v)
        _detile_chunk(tail_v, out_v.at[0], iotas, n_tail)
        pltpu.sync_copy(out_v.at[0, pl.ds(0, n_tail)],
                        t2.at[pl.ds(_NCH * _CB, n_tail)])


def _gather_block(table2, ids_t, idx_v, rows_v, gsem, k, b, col0):
    pltpu.sync_copy(ids_t.at[k, pl.ds(col0, _CB)], idx_v.at[b])
    pltpu.async_copy(table2.at[idx_v.at[b]], rows_v.at[b], gsem)


def _emb_kernel(ids_t, table2, pos_hbm, out_hbm,
                idx_v, rows_v, outb_v, pos_v,
                gsem0, gsem1, osem0, osem1):
    w = lax.axis_index("s") * _NC + lax.axis_index("c")
    col0 = w * _CB
    gsems = (gsem0, gsem1)
    osems = (osem0, osem1)

    pltpu.sync_copy(pos_hbm, pos_v)
    iota = lax.iota(jnp.int32, 16)
    fidxs = [iota + i * 16 for i in range(_D // 16)]

    def out_slice(k):
        return out_hbm.at[k, :, pl.ds(col0, _CB)]

    _gather_block(table2, ids_t, idx_v, rows_v, gsems[0], 0, 0, col0)

    def outer(kk, carry):
        for b in range(2):
            k = 2 * kk + b
            pltpu.make_async_copy(table2.at[idx_v.at[b]], rows_v.at[b],
                                  gsems[b]).wait()

            @pl.when(k + 1 < _SEQ)
            def _():
                _gather_block(table2, ids_t, idx_v, rows_v,
                              gsems[1 - b], k + 1, 1 - b, col0)

            @pl.when(k >= 2)
            def _():
                pltpu.make_async_copy(outb_v.at[b], out_slice(k), osems[b]).wait()

            pos4 = [pos_v[pl.ds(k * _D + i * 16, 16)] for i in range(_D // 16)]

            ni = _D // 16
            tu = 4

            def tbody(t4, carry2):
                t0 = t4 * tu
                tsplats = [jnp.full((16,), t0 + d, jnp.int32)
                           for d in range(tu)]
                vals = [rows_v[b, t0 + d, pl.ds(i * 16, 16)]
                        for d in range(tu) for i in range(ni)]
                sums = [vals[d * ni + i] + pos4[i]
                        for d in range(tu) for i in range(ni)]
                for d in range(tu):
                    for i in range(ni):
                        plsc.store_scatter(outb_v.at[b], [fidxs[i], tsplats[d]],
                                           sums[d * ni + i])
                return carry2

            lax.fori_loop(0, _CB // tu, tbody, 0)

            pltpu.async_copy(outb_v.at[b], out_slice(k), osems[b])
        return carry

    lax.fori_loop(0, _SEQ // 2, outer, 0)
    for b in range(2):
        pltpu.make_async_copy(outb_v.at[b], out_slice(b), osems[b]).wait()


@jax.jit
def _run(ids_t, tt, pos_flat):
    mesh = plsc.VectorSubcoreMesh(core_axis_name="c", subcore_axis_name="s")
    table2 = pl.kernel(
        _relayout_kernel,
        out_type=jax.ShapeDtypeStruct((_VOCAB, 2 * _D), jnp.float32),
        mesh=mesh,
        scratch_types=[
            pltpu.VMEM((2, _D, _CB), jnp.float32),
            pltpu.VMEM((2, _CB, 2 * _D), jnp.float32),
            pltpu.VMEM((_D, _VOCAB - _NCH * _CB), jnp.float32),
            pltpu.SemaphoreType.DMA,
            pltpu.SemaphoreType.DMA,
            pltpu.SemaphoreType.DMA,
            pltpu.SemaphoreType.DMA,
        ],
        compiler_params=pltpu.CompilerParams(needs_layout_passes=False),
    )(tt)
    return pl.kernel(
        _emb_kernel,
        out_type=jax.ShapeDtypeStruct((_SEQ, _D, _BATCH), jnp.float32),
        mesh=mesh,
        scratch_types=[
            pltpu.VMEM((2, _CB), jnp.int32),
            pltpu.VMEM((2, _CB, 2 * _D), jnp.float32),
            pltpu.VMEM((2, _D, _CB), jnp.float32),
            pltpu.VMEM((_SEQ * _D,), jnp.float32),
            pltpu.SemaphoreType.DMA,
            pltpu.SemaphoreType.DMA,
            pltpu.SemaphoreType.DMA,
            pltpu.SemaphoreType.DMA,
        ],
        compiler_params=pltpu.CompilerParams(needs_layout_passes=False),
    )(ids_t, table2, pos_flat)


def kernel(token_ids, token_table, pos_table):
    ids_t = token_ids.astype(jnp.int32).T
    tt = token_table.T
    out = _run(ids_t, tt, pos_table.reshape(-1))
    return jnp.transpose(out, (2, 0, 1))


# speedup vs baseline: 2.3013x; 2.3013x over previous
import functools

import jax
import jax.numpy as jnp
from jax import lax
from jax.experimental import pallas as pl
from jax.experimental.pallas import tpu as pltpu
from jax.experimental.pallas import tpu_sc as plsc

_NC = 2
_NS = 16
_NW = _NC * _NS

_VOCAB = 1000000
_D = 64
_SEQ = 200
_BATCH = 4096
_CB = 128
_VR = _VOCAB // 2
_NCH = _VOCAB // _CB
_CH_MAIN = (_NCH // _NW) * _NW


def _detile_chunk(in_b, out_b, iota, n_tok):
    def tgbody(tg, carry):
        t0v = tg * 16 + iota
        for f0 in range(0, _D, 16):
            fv = f0 + iota
            vals = []
            tds = []
            for d in range(0, 16, 4):
                for dd in range(4):
                    td = tg * 16 + ((iota + d + dd) & 15)
                    tds.append(td)
                    vals.append(plsc.load_gather(in_b, [fv, td]))
            for j in range(16):
                td = tds[j]
                plsc.store_scatter(out_b,
                                   [lax.shift_right_logical(td, 1),
                                    ((td & 1) << 6) + fv],
                                   vals[j])
        return carry

    lax.fori_loop(0, n_tok // 16, tgbody, 0)


def _relayout_kernel(tt, t2, in_v, out_v, tail_v, isem0, isem1, osem0, osem1):
    w = lax.axis_index("s") * _NC + lax.axis_index("c")
    isems = (isem0, isem1)
    osems = (osem0, osem1)
    iota = lax.iota(jnp.int32, 16)

    def in_slice(c):
        return tt.at[:, pl.ds(c * _CB, _CB)]

    def out_slice(c):
        return t2.at[pl.ds(c * (_CB // 2), _CB // 2)]

    pltpu.async_copy(in_slice(w), in_v.at[0], isems[0])

    def outer(kk, carry):
        for b in range(2):
            j = 2 * kk + b
            c = w + j * _NW
            pltpu.make_async_copy(in_slice(c), in_v.at[b], isems[b]).wait()

            @pl.when(c + _NW < _CH_MAIN)
            def _():
                pltpu.async_copy(in_slice(c + _NW), in_v.at[1 - b], isems[1 - b])

            @pl.when(j >= 2)
            def _():
                pltpu.make_async_copy(out_v.at[b], out_slice(c), osems[b]).wait()

            _detile_chunk(in_v.at[b], out_v.at[b], iota, _CB)
            pltpu.async_copy(out_v.at[b], out_slice(c), osems[b])
        return carry

    lax.fori_loop(0, _CH_MAIN // _NW // 2, outer, 0)
    for b in range(2):
        pltpu.make_async_copy(out_v.at[b], out_slice(0), osems[b]).wait()

    c_tail = w + _CH_MAIN

    @pl.when(c_tail < _NCH)
    def _():
        pltpu.sync_copy(in_slice(c_tail), in_v.at[0])
        _detile_chunk(in_v.at[0], out_v.at[0], iota, _CB)
        pltpu.sync_copy(out_v.at[0], out_slice(c_tail))

    @pl.when(c_tail == _NCH)
    def _():
        n_tail = _VOCAB - _NCH * _CB
        pltpu.sync_copy(tt.at[:, pl.ds(_NCH * _CB, n_tail)], tail_v)
        _detile_chunk(tail_v, out_v.at[0], iota, n_tail)
        pltpu.sync_copy(out_v.at[0, pl.ds(0, n_tail // 2)],
                        t2.at[pl.ds(_NCH * (_CB // 2), n_tail // 2)])


def _gather_block(table2, ids_t, idx_raw_v, idx2_v, rows_v, gsem, k, b, col0):
    pltpu.sync_copy(ids_t.at[k, pl.ds(col0, _CB)], idx_raw_v.at[b])
    for t in range(_CB // 16):
        sl = pl.ds(t * 16, 16)
        idx2_v[b, sl] = lax.shift_right_logical(idx_raw_v[b, sl], 1)
    pltpu.async_copy(table2.at[idx2_v.at[b]], rows_v.at[b], gsem)


def _emb_kernel(ids_t, table2, pos_hbm, out_hbm,
                idx_raw_v, idx2_v, rows_v, outb_v, pos_v,
                gsem0, gsem1, osem0, osem1):
    w = lax.axis_index("s") * _NC + lax.axis_index("c")
    col0 = w * _CB
    gsems = (gsem0, gsem1)
    osems = (osem0, osem1)

    pltpu.sync_copy(pos_hbm, pos_v)
    iota = lax.iota(jnp.int32, 16)

    def out_slice(k):
        return out_hbm.at[k, :, pl.ds(col0, _CB)]

    _gather_block(table2, ids_t, idx_raw_v, idx2_v, rows_v, gsems[0],
                  0, 0, col0)

    def outer(kk, carry):
        for b in range(2):
            k = 2 * kk + b
            pltpu.make_async_copy(table2.at[idx2_v.at[b]], rows_v.at[b],
                                  gsems[b]).wait()

            @pl.when(k + 1 < _SEQ)
            def _():
                _gather_block(table2, ids_t, idx_raw_v, idx2_v, rows_v,
                              gsems[1 - b], k + 1, 1 - b, col0)

            @pl.when(k >= 2)
            def _():
                pltpu.make_async_copy(outb_v.at[b], out_slice(k), osems[b]).wait()

            kbase = jnp.full((16,), k * _D, jnp.int32)

            def tgbody(tg, carry2):
                t0v = tg * 16 + iota
                parv = (idx_raw_v[b, pl.ds(tg * 16, 16)] & 1) << 6
                for f0 in range(0, _D, 16):
                    vals = []
                    sums = []
                    fms = []
                    for d in range(16):
                        fm = f0 + ((iota + d) & 15)
                        fms.append(fm)
                        v = plsc.load_gather(rows_v.at[b], [t0v, parv + fm])
                        p = plsc.load_gather(pos_v, [kbase + fm])
                        vals.append((v, p))
                    for j in range(16):
                        v, p = vals[j]
                        plsc.store_scatter(outb_v.at[b], [fms[j], t0v], v + p)
                return carry2

            lax.fori_loop(0, _CB // 16, tgbody, 0)

            pltpu.async_copy(outb_v.at[b], out_slice(k), osems[b])
        return carry

    lax.fori_loop(0, _SEQ // 2, outer, 0)
    for b in range(2):
        pltpu.make_async_copy(outb_v.at[b], out_slice(b), osems[b]).wait()


@jax.jit
def _run(ids_t, tt, pos_flat):
    mesh = plsc.VectorSubcoreMesh(core_axis_name="c", subcore_axis_name="s")
    table2 = pl.kernel(
        _relayout_kernel,
        out_type=jax.ShapeDtypeStruct((_VR, 2 * _D), jnp.float32),
        mesh=mesh,
        scratch_types=[
            pltpu.VMEM((2, _D, _CB), jnp.float32),
            pltpu.VMEM((2, _CB // 2, 2 * _D), jnp.float32),
            pltpu.VMEM((_D, _VOCAB - _NCH * _CB), jnp.float32),
            pltpu.SemaphoreType.DMA,
            pltpu.SemaphoreType.DMA,
            pltpu.SemaphoreType.DMA,
            pltpu.SemaphoreType.DMA,
        ],
        compiler_params=pltpu.CompilerParams(needs_layout_passes=False),
    )(tt)
    return pl.kernel(
        _emb_kernel,
        out_type=jax.ShapeDtypeStruct((_SEQ, _D, _BATCH), jnp.float32),
        mesh=mesh,
        scratch_types=[
            pltpu.VMEM((2, _CB), jnp.int32),
            pltpu.VMEM((2, _CB), jnp.int32),
            pltpu.VMEM((2, _CB, 2 * _D), jnp.float32),
            pltpu.VMEM((2, _D, _CB), jnp.float32),
            pltpu.VMEM((_SEQ * _D,), jnp.float32),
            pltpu.SemaphoreType.DMA,
            pltpu.SemaphoreType.DMA,
            pltpu.SemaphoreType.DMA,
            pltpu.SemaphoreType.DMA,
        ],
        compiler_params=pltpu.CompilerParams(needs_layout_passes=False),
    )(ids_t, table2, pos_flat)


def kernel(token_ids, token_table, pos_table):
    ids_t = token_ids.astype(jnp.int32).T
    tt = token_table.T
    out = _run(ids_t, tt, pos_table.reshape(-1))
    return jnp.transpose(out, (2, 0, 1))


# speedup vs baseline: 3.0655x; 1.3321x over previous
import functools

import jax
import jax.numpy as jnp
from jax import lax
from jax.experimental import pallas as pl
from jax.experimental.pallas import tpu as pltpu
from jax.experimental.pallas import tpu_sc as plsc

_NC = 2
_NS = 16
_NW = _NC * _NS

_VOCAB = 1000000
_D = 64
_SEQ = 200
_BATCH = 4096
_CB = 128
_VR = _VOCAB // 2
_NCH = _VOCAB // _CB
_CH_MAIN = (_NCH // _NW) * _NW


def _detile_chunk(in_b, out_b, iota, n_tok):
    def tgbody(tg, carry):
        t0v = tg * 16 + iota
        for f0 in range(0, _D, 16):
            fv = f0 + iota
            vals = []
            tds = []
            for d in range(0, 16, 4):
                for dd in range(4):
                    td = tg * 16 + ((iota + d + dd) & 15)
                    tds.append(td)
                    vals.append(plsc.load_gather(in_b, [fv, td]))
            for j in range(16):
                td = tds[j]
                plsc.store_scatter(out_b,
                                   [lax.shift_right_logical(td, 1),
                                    ((td & 1) << 6) + fv],
                                   vals[j])
        return carry

    lax.fori_loop(0, n_tok // 16, tgbody, 0)


def _relayout_kernel(tt, t2, in_v, out_v, tail_v, isem0, isem1, osem0, osem1):
    w = lax.axis_index("s") * _NC + lax.axis_index("c")
    isems = (isem0, isem1)
    osems = (osem0, osem1)
    iota = lax.iota(jnp.int32, 16)

    def in_slice(c):
        return tt.at[:, pl.ds(c * _CB, _CB)]

    def out_slice(c):
        return t2.at[pl.ds(c * (_CB // 2), _CB // 2)]

    pltpu.async_copy(in_slice(w), in_v.at[0], isems[0])

    def outer(kk, carry):
        for b in range(2):
            j = 2 * kk + b
            c = w + j * _NW
            pltpu.make_async_copy(in_slice(c), in_v.at[b], isems[b]).wait()

            @pl.when(c + _NW < _CH_MAIN)
            def _():
                pltpu.async_copy(in_slice(c + _NW), in_v.at[1 - b], isems[1 - b])

            @pl.when(j >= 2)
            def _():
                pltpu.make_async_copy(out_v.at[b], out_slice(c), osems[b]).wait()

            _detile_chunk(in_v.at[b], out_v.at[b], iota, _CB)
            pltpu.async_copy(out_v.at[b], out_slice(c), osems[b])
        return carry

    lax.fori_loop(0, _CH_MAIN // _NW // 2, outer, 0)
    for b in range(2):
        pltpu.make_async_copy(out_v.at[b], out_slice(0), osems[b]).wait()

    c_tail = w + _CH_MAIN

    @pl.when(c_tail < _NCH)
    def _():
        pltpu.sync_copy(in_slice(c_tail), in_v.at[0])
        _detile_chunk(in_v.at[0], out_v.at[0], iota, _CB)
        pltpu.sync_copy(out_v.at[0], out_slice(c_tail))

    @pl.when(c_tail == _NCH)
    def _():
        n_tail = _VOCAB - _NCH * _CB
        pltpu.sync_copy(tt.at[:, pl.ds(_NCH * _CB, n_tail)], tail_v)
        _detile_chunk(tail_v, out_v.at[0], iota, n_tail)
        pltpu.sync_copy(out_v.at[0, pl.ds(0, n_tail // 2)],
                        t2.at[pl.ds(_NCH * (_CB // 2), n_tail // 2)])


def _gather_block(table2, ids_t, idx_raw_v, idx2_v, rows_v, gsem, k, b, col0):
    pltpu.sync_copy(ids_t.at[k, pl.ds(col0, _CB)], idx_raw_v.at[b])
    for t in range(_CB // 16):
        sl = pl.ds(t * 16, 16)
        idx2_v[b, sl] = lax.shift_right_logical(idx_raw_v[b, sl], 1)
    pltpu.async_copy(table2.at[idx2_v.at[b]], rows_v.at[b], gsem)


def _emb_kernel(ids_t, table2, pos_hbm, out_hbm,
                idx_raw_v, idx2_v, rows_v, outb_v, pos_v,
                gsem0, gsem1, osem0, osem1):
    w = lax.axis_index("s") * _NC + lax.axis_index("c")
    col0 = w * _CB
    gsems = (gsem0, gsem1)
    osems = (osem0, osem1)

    pltpu.sync_copy(pos_hbm, pos_v)
    iota = lax.iota(jnp.int32, 16)

    def out_slice(k):
        return out_hbm.at[k, :, pl.ds(col0, _CB)]

    _gather_block(table2, ids_t, idx_raw_v, idx2_v, rows_v, gsems[0],
                  0, 0, col0)

    def outer(kk, carry):
        for b in range(2):
            k = 2 * kk + b
            pltpu.make_async_copy(table2.at[idx2_v.at[b]], rows_v.at[b],
                                  gsems[b]).wait()

            @pl.when(k + 1 < _SEQ)
            def _():
                _gather_block(table2, ids_t, idx_raw_v, idx2_v, rows_v,
                              gsems[1 - b], k + 1, 1 - b, col0)

            @pl.when(k >= 2)
            def _():
                pltpu.make_async_copy(outb_v.at[b], out_slice(k), osems[b]).wait()

            kbase = jnp.full((16,), k * _D, jnp.int32)

            for f0 in range(0, _D, 16):
                fms0 = [f0 + ((iota + d) & 15) for d in range(16)]
                pdiag0 = tuple(plsc.load_gather(pos_v, [kbase + fms0[d]])
                               for d in range(16))

                def tgbody(tg, pdiag, f0=f0):
                    t0v = tg * 16 + iota
                    parv = (idx_raw_v[b, pl.ds(tg * 16, 16)] & 1) << 6
                    vals = []
                    fms = []
                    for d in range(16):
                        fm = f0 + ((iota + d) & 15)
                        fms.append(fm)
                        vals.append(plsc.load_gather(rows_v.at[b],
                                                     [t0v, parv + fm]))
                    for d in range(16):
                        plsc.store_scatter(outb_v.at[b], [fms[d], t0v],
                                           vals[d] + pdiag[d])
                    return pdiag

                lax.fori_loop(0, _CB // 16, tgbody, pdiag0)

            pltpu.async_copy(outb_v.at[b], out_slice(k), osems[b])
        return carry

    lax.fori_loop(0, _SEQ // 2, outer, 0)
    for b in range(2):
        pltpu.make_async_copy(outb_v.at[b], out_slice(b), osems[b]).wait()


@jax.jit
def _run(ids_t, tt, pos_flat):
    mesh = plsc.VectorSubcoreMesh(core_axis_name="c", subcore_axis_name="s")
    table2 = pl.kernel(
        _relayout_kernel,
        out_type=jax.ShapeDtypeStruct((_VR, 2 * _D), jnp.float32),
        mesh=mesh,
        scratch_types=[
            pltpu.VMEM((2, _D, _CB), jnp.float32),
            pltpu.VMEM((2, _CB // 2, 2 * _D), jnp.float32),
            pltpu.VMEM((_D, _VOCAB - _NCH * _CB), jnp.float32),
            pltpu.SemaphoreType.DMA,
            pltpu.SemaphoreType.DMA,
            pltpu.SemaphoreType.DMA,
            pltpu.SemaphoreType.DMA,
        ],
        compiler_params=pltpu.CompilerParams(needs_layout_passes=False),
    )(tt)
    return pl.kernel(
        _emb_kernel,
        out_type=jax.ShapeDtypeStruct((_SEQ, _D, _BATCH), jnp.float32),
        mesh=mesh,
        scratch_types=[
            pltpu.VMEM((2, _CB), jnp.int32),
            pltpu.VMEM((2, _CB), jnp.int32),
            pltpu.VMEM((2, _CB, 2 * _D), jnp.float32),
            pltpu.VMEM((2, _D, _CB), jnp.float32),
            pltpu.VMEM((_SEQ * _D,), jnp.float32),
            pltpu.SemaphoreType.DMA,
            pltpu.SemaphoreType.DMA,
            pltpu.SemaphoreType.DMA,
            pltpu.SemaphoreType.DMA,
        ],
        compiler_params=pltpu.CompilerParams(needs_layout_passes=False),
    )(ids_t, table2, pos_flat)


def kernel(token_ids, token_table, pos_table):
    ids_t = token_ids.astype(jnp.int32).T
    tt = token_table.T
    out = _run(ids_t, tt, pos_table.reshape(-1))
    return jnp.transpose(out, (2, 0, 1))


# speedup vs baseline: 3.6313x; 1.1846x over previous
import functools

import jax
import jax.numpy as jnp
from jax import lax
from jax.experimental import pallas as pl
from jax.experimental.pallas import tpu as pltpu
from jax.experimental.pallas import tpu_sc as plsc

_NC = 2
_NS = 16
_NW = _NC * _NS

_VOCAB = 1000000
_D = 64
_SEQ = 200
_BATCH = 4096
_CB = 128
_TB = 256
_NBB = _BATCH // _TB
_SPW = _SEQ // (_NW // _NBB)
_VR = _VOCAB // 2
_NCH = _VOCAB // _CB
_CH_MAIN = (_NCH // _NW) * _NW


def _detile_chunk(in_b, out_b, iota, n_tok):
    def tgbody(tg, carry):
        t0v = tg * 16 + iota
        for f0 in range(0, _D, 16):
            fv = f0 + iota
            vals = []
            tds = []
            for d in range(0, 16, 4):
                for dd in range(4):
                    td = tg * 16 + ((iota + d + dd) & 15)
                    tds.append(td)
                    vals.append(plsc.load_gather(in_b, [fv, td]))
            for j in range(16):
                td = tds[j]
                plsc.store_scatter(out_b,
                                   [lax.shift_right_logical(td, 1),
                                    ((td & 1) << 6) + fv],
                                   vals[j])
        return carry

    lax.fori_loop(0, n_tok // 16, tgbody, 0)


def _relayout_kernel(tt, t2, in_v, out_v, tail_v, isem0, isem1, osem0, osem1):
    w = lax.axis_index("s") * _NC + lax.axis_index("c")
    isems = (isem0, isem1)
    osems = (osem0, osem1)
    iota = lax.iota(jnp.int32, 16)

    def in_slice(c):
        return tt.at[:, pl.ds(c * _CB, _CB)]

    def out_slice(c):
        return t2.at[pl.ds(c * (_CB // 2), _CB // 2)]

    pltpu.async_copy(in_slice(w), in_v.at[0], isems[0])

    def outer(kk, carry):
        for b in range(2):
            j = 2 * kk + b
            c = w + j * _NW
            pltpu.make_async_copy(in_slice(c), in_v.at[b], isems[b]).wait()

            @pl.when(c + _NW < _CH_MAIN)
            def _():
                pltpu.async_copy(in_slice(c + _NW), in_v.at[1 - b], isems[1 - b])

            @pl.when(j >= 2)
            def _():
                pltpu.make_async_copy(out_v.at[b], out_slice(c), osems[b]).wait()

            _detile_chunk(in_v.at[b], out_v.at[b], iota, _CB)
            pltpu.async_copy(out_v.at[b], out_slice(c), osems[b])
        return carry

    lax.fori_loop(0, _CH_MAIN // _NW // 2, outer, 0)
    for b in range(2):
        pltpu.make_async_copy(out_v.at[b], out_slice(0), osems[b]).wait()

    c_tail = w + _CH_MAIN

    @pl.when(c_tail < _NCH)
    def _():
        pltpu.sync_copy(in_slice(c_tail), in_v.at[0])
        _detile_chunk(in_v.at[0], out_v.at[0], iota, _CB)
        pltpu.sync_copy(out_v.at[0], out_slice(c_tail))

    @pl.when(c_tail == _NCH)
    def _():
        n_tail = _VOCAB - _NCH * _CB
        pltpu.sync_copy(tt.at[:, pl.ds(_NCH * _CB, n_tail)], tail_v)
        _detile_chunk(tail_v, out_v.at[0], iota, n_tail)
        pltpu.sync_copy(out_v.at[0, pl.ds(0, n_tail // 2)],
                        t2.at[pl.ds(_NCH * (_CB // 2), n_tail // 2)])


def _gather_block(table2, ids_t, idx_raw_v, idx2_v, rows_v, gsem, k, b, col0):
    pltpu.sync_copy(ids_t.at[k, pl.ds(col0, _TB)], idx_raw_v.at[b])
    for t in range(_TB // 16):
        h, sl2 = t // 8, pl.ds((t % 8) * 16, 16)
        idx2_v[b, h, sl2] = lax.shift_right_logical(
            idx_raw_v[b, pl.ds(t * 16, 16)], 1)
    for h in range(2):
        pltpu.async_copy(table2.at[idx2_v.at[b, h]],
                         rows_v.at[b, pl.ds(h * 128, 128)], gsem)


def _wait_gathers(table2, idx2_v, rows_v, gsem, b):
    for h in range(2):
        pltpu.make_async_copy(table2.at[idx2_v.at[b, h]],
                              rows_v.at[b, pl.ds(h * 128, 128)], gsem).wait()


def _emb_kernel(ids_t, table2, pos_hbm, out_hbm,
                idx_raw_v, idx2_v, rows_v, outb_v, pos_v,
                gsem0, gsem1, osem0, osem1):
    w = lax.axis_index("s") * _NC + lax.axis_index("c")
    col0 = (w % _NBB) * _TB
    s_base = (w // _NBB) * (_SEQ // (_NW // _NBB))
    gsems = (gsem0, gsem1)
    osems = (osem0, osem1)

    pltpu.sync_copy(pos_hbm, pos_v)
    iota = lax.iota(jnp.int32, 16)

    def out_slice(k):
        return out_hbm.at[k, :, pl.ds(col0, _TB)]

    _gather_block(table2, ids_t, idx_raw_v, idx2_v, rows_v, gsems[0],
                  s_base, 0, col0)

    def outer(kk, carry):
        for b in range(2):
            k = s_base + 2 * kk + b
            _wait_gathers(table2, idx2_v, rows_v, gsems[b], b)

            @pl.when(k + 1 < s_base + _SPW)
            def _():
                _gather_block(table2, ids_t, idx_raw_v, idx2_v, rows_v,
                              gsems[1 - b], k + 1, 1 - b, col0)

            @pl.when(k >= s_base + 2)
            def _():
                pltpu.make_async_copy(outb_v.at[b], out_slice(k), osems[b]).wait()

            kbase = jnp.full((16,), k * _D, jnp.int32)

            for f0 in range(0, _D, 16):
                fms0 = [f0 + ((iota + d) & 15) for d in range(16)]
                pdiag0 = tuple(plsc.load_gather(pos_v, [kbase + fms0[d]])
                               for d in range(16))

                def tgbody(tg, pdiag, f0=f0):
                    t0v = tg * 16 + iota
                    parv = (idx_raw_v[b, pl.ds(tg * 16, 16)] & 1) << 6
                    vals = []
                    fms = []
                    for d in range(16):
                        fm = f0 + ((iota + d) & 15)
                        fms.append(fm)
                        vals.append(plsc.load_gather(rows_v.at[b],
                                                     [t0v, parv + fm]))
                    for d in range(16):
                        plsc.store_scatter(outb_v.at[b], [fms[d], t0v],
                                           vals[d] + pdiag[d])
                    return pdiag

                lax.fori_loop(0, _TB // 16, tgbody, pdiag0)

            pltpu.async_copy(outb_v.at[b], out_slice(k), osems[b])
        return carry

    lax.fori_loop(0, _SPW // 2, outer, 0)
    for b in range(2):
        pltpu.make_async_copy(outb_v.at[b], out_slice(s_base + b),
                              osems[b]).wait()


@jax.jit
def _run(ids_t, tt, pos_flat):
    mesh = plsc.VectorSubcoreMesh(core_axis_name="c", subcore_axis_name="s")
    table2 = pl.kernel(
        _relayout_kernel,
        out_type=jax.ShapeDtypeStruct((_VR, 2 * _D), jnp.float32),
        mesh=mesh,
        scratch_types=[
            pltpu.VMEM((2, _D, _CB), jnp.float32),
            pltpu.VMEM((2, _CB // 2, 2 * _D), jnp.float32),
            pltpu.VMEM((_D, _VOCAB - _NCH * _CB), jnp.float32),
            pltpu.SemaphoreType.DMA,
            pltpu.SemaphoreType.DMA,
            pltpu.SemaphoreType.DMA,
            pltpu.SemaphoreType.DMA,
        ],
        compiler_params=pltpu.CompilerParams(needs_layout_passes=False),
    )(tt)
    return pl.kernel(
        _emb_kernel,
        out_type=jax.ShapeDtypeStruct((_SEQ, _D, _BATCH), jnp.float32),
        mesh=mesh,
        scratch_types=[
            pltpu.VMEM((2, _TB), jnp.int32),
            pltpu.VMEM((2, 2, 128), jnp.int32),
            pltpu.VMEM((2, _TB, 2 * _D), jnp.float32),
            pltpu.VMEM((2, _D, _TB), jnp.float32),
            pltpu.VMEM((_SEQ * _D,), jnp.float32),
            pltpu.SemaphoreType.DMA,
            pltpu.SemaphoreType.DMA,
            pltpu.SemaphoreType.DMA,
            pltpu.SemaphoreType.DMA,
        ],
        compiler_params=pltpu.CompilerParams(needs_layout_passes=False),
    )(ids_t, table2, pos_flat)


def kernel(token_ids, token_table, pos_table):
    ids_t = token_ids.astype(jnp.int32).T
    tt = token_table.T
    out = _run(ids_t, tt, pos_table.reshape(-1))
    return jnp.transpose(out, (2, 0, 1))


# speedup vs baseline: 4.2287x; 1.1645x over previous
import functools

import jax
import jax.numpy as jnp
from jax import lax
from jax.experimental import pallas as pl
from jax.experimental.pallas import tpu as pltpu
from jax.experimental.pallas import tpu_sc as plsc

_NC = 2
_NS = 16
_NW = _NC * _NS

_VOCAB = 1000000
_D = 64
_SEQ = 200
_BATCH = 4096
_CBA = 256
_TB = 256
_NBB = _BATCH // _TB
_SPW = _SEQ // (_NW // _NBB)
_VR = _VOCAB // 2
_NCH = _VOCAB // _CBA
_CH_MAIN = (_NCH // _NW) * _NW


def _detile_chunk(in_b, out_b, iota, n_tok):
    def tgbody(tg, carry):
        t0v = tg * 16 + iota
        for f0 in range(0, _D, 16):
            fv = f0 + iota
            vals = []
            tds = []
            for d in range(0, 16, 4):
                for dd in range(4):
                    td = tg * 16 + ((iota + d + dd) & 15)
                    tds.append(td)
                    vals.append(plsc.load_gather(in_b, [fv, td]))
            for j in range(16):
                td = tds[j]
                plsc.store_scatter(out_b,
                                   [lax.shift_right_logical(td, 1),
                                    ((td & 1) << 6) + fv],
                                   vals[j])
        return carry

    lax.fori_loop(0, n_tok // 16, tgbody, 0)


def _relayout_kernel(tt, t2, in_v, out_v, tail_v, isem0, isem1, osem0, osem1):
    w = lax.axis_index("s") * _NC + lax.axis_index("c")
    isems = (isem0, isem1)
    osems = (osem0, osem1)
    iota = lax.iota(jnp.int32, 16)

    def in_slice(c):
        return tt.at[:, pl.ds(c * _CBA, _CBA)]

    def out_slice(c):
        return t2.at[pl.ds(c * (_CBA // 2), _CBA // 2)]

    pltpu.async_copy(in_slice(w), in_v.at[0], isems[0])

    def outer(kk, carry):
        for b in range(2):
            j = 2 * kk + b
            c = w + j * _NW
            pltpu.make_async_copy(in_slice(c), in_v.at[b], isems[b]).wait()

            @pl.when(c + _NW < _CH_MAIN)
            def _():
                pltpu.async_copy(in_slice(c + _NW), in_v.at[1 - b], isems[1 - b])

            @pl.when(j >= 2)
            def _():
                pltpu.make_async_copy(out_v.at[b], out_slice(c), osems[b]).wait()

            _detile_chunk(in_v.at[b], out_v.at[b], iota, _CBA)
            pltpu.async_copy(out_v.at[b], out_slice(c), osems[b])
        return carry

    lax.fori_loop(0, _CH_MAIN // _NW // 2, outer, 0)
    for b in range(2):
        pltpu.make_async_copy(out_v.at[b], out_slice(0), osems[b]).wait()

    c_tail = w + _CH_MAIN

    @pl.when(c_tail < _NCH)
    def _():
        pltpu.sync_copy(in_slice(c_tail), in_v.at[0])
        _detile_chunk(in_v.at[0], out_v.at[0], iota, _CBA)
        pltpu.sync_copy(out_v.at[0], out_slice(c_tail))

    @pl.when(c_tail == _NCH)
    def _():
        n_tail = _VOCAB - _NCH * _CBA
        pltpu.sync_copy(tt.at[:, pl.ds(_NCH * _CBA, n_tail)], tail_v)
        _detile_chunk(tail_v, out_v.at[0], iota, n_tail)
        pltpu.sync_copy(out_v.at[0, pl.ds(0, n_tail // 2)],
                        t2.at[pl.ds(_NCH * (_CBA // 2), n_tail // 2)])


def _gather_block(table2, ids_t, idx_raw_v, idx2_v, rows_v, gsem, k, b, col0):
    pltpu.sync_copy(ids_t.at[k, pl.ds(col0, _TB)], idx_raw_v.at[b])
    for t in range(_TB // 16):
        h, sl2 = t // 8, pl.ds((t % 8) * 16, 16)
        idx2_v[b, h, sl2] = lax.shift_right_logical(
            idx_raw_v[b, pl.ds(t * 16, 16)], 1)
    for h in range(2):
        pltpu.async_copy(table2.at[idx2_v.at[b, h]],
                         rows_v.at[b, pl.ds(h * 128, 128)], gsem)


def _wait_gathers(table2, idx2_v, rows_v, gsem, b):
    for h in range(2):
        pltpu.make_async_copy(table2.at[idx2_v.at[b, h]],
                              rows_v.at[b, pl.ds(h * 128, 128)], gsem).wait()


def _emb_kernel(ids_t, table2, pos_hbm, out_hbm,
                idx_raw_v, idx2_v, rows_v, outb_v, pos_v,
                gsem0, gsem1, osem0, osem1):
    w = lax.axis_index("s") * _NC + lax.axis_index("c")
    col0 = (w % _NBB) * _TB
    s_base = (w // _NBB) * (_SEQ // (_NW // _NBB))
    gsems = (gsem0, gsem1)
    osems = (osem0, osem1)

    pltpu.sync_copy(pos_hbm, pos_v)
    iota = lax.iota(jnp.int32, 16)

    def out_slice(k):
        return out_hbm.at[k, :, pl.ds(col0, _TB)]

    _gather_block(table2, ids_t, idx_raw_v, idx2_v, rows_v, gsems[0],
                  s_base, 0, col0)

    def outer(kk, carry):
        for b in range(2):
            k = s_base + 2 * kk + b
            _wait_gathers(table2, idx2_v, rows_v, gsems[b], b)

            @pl.when(k + 1 < s_base + _SPW)
            def _():
                _gather_block(table2, ids_t, idx_raw_v, idx2_v, rows_v,
                              gsems[1 - b], k + 1, 1 - b, col0)

            @pl.when(k >= s_base + 2)
            def _():
                pltpu.make_async_copy(outb_v.at[b], out_slice(k), osems[b]).wait()

            kbase = jnp.full((16,), k * _D, jnp.int32)

            for f0 in range(0, _D, 16):
                fms0 = [f0 + ((iota + d) & 15) for d in range(16)]
                pdiag0 = tuple(plsc.load_gather(pos_v, [kbase + fms0[d]])
                               for d in range(16))

                def tgbody(tg, pdiag, f0=f0):
                    t0v = tg * 16 + iota
                    parv = (idx_raw_v[b, pl.ds(tg * 16, 16)] & 1) << 6
                    vals = []
                    fms = []
                    for d in range(16):
                        fm = f0 + ((iota + d) & 15)
                        fms.append(fm)
                        vals.append(plsc.load_gather(rows_v.at[b],
                                                     [t0v, parv + fm]))
                    for d in range(16):
                        plsc.store_scatter(outb_v.at[b], [fms[d], t0v],
                                           vals[d] + pdiag[d])
                    return pdiag

                lax.fori_loop(0, _TB // 16, tgbody, pdiag0)

            pltpu.async_copy(outb_v.at[b], out_slice(k), osems[b])
        return carry

    lax.fori_loop(0, _SPW // 2, outer, 0)
    for b in range(2):
        pltpu.make_async_copy(outb_v.at[b], out_slice(s_base + b),
                              osems[b]).wait()


@jax.jit
def _run(ids_t, tt, pos_flat):
    mesh = plsc.VectorSubcoreMesh(core_axis_name="c", subcore_axis_name="s")
    table2 = pl.kernel(
        _relayout_kernel,
        out_type=jax.ShapeDtypeStruct((_VR, 2 * _D), jnp.float32),
        mesh=mesh,
        scratch_types=[
            pltpu.VMEM((2, _D, _CBA), jnp.float32),
            pltpu.VMEM((2, _CBA // 2, 2 * _D), jnp.float32),
            pltpu.VMEM((_D, _VOCAB - _NCH * _CBA), jnp.float32),
            pltpu.SemaphoreType.DMA,
            pltpu.SemaphoreType.DMA,
            pltpu.SemaphoreType.DMA,
            pltpu.SemaphoreType.DMA,
        ],
        compiler_params=pltpu.CompilerParams(needs_layout_passes=False),
    )(tt)
    return pl.kernel(
        _emb_kernel,
        out_type=jax.ShapeDtypeStruct((_SEQ, _D, _BATCH), jnp.float32),
        mesh=mesh,
        scratch_types=[
            pltpu.VMEM((2, _TB), jnp.int32),
            pltpu.VMEM((2, 2, 128), jnp.int32),
            pltpu.VMEM((2, _TB, 2 * _D), jnp.float32),
            pltpu.VMEM((2, _D, _TB), jnp.float32),
            pltpu.VMEM((_SEQ * _D,), jnp.float32),
            pltpu.SemaphoreType.DMA,
            pltpu.SemaphoreType.DMA,
            pltpu.SemaphoreType.DMA,
            pltpu.SemaphoreType.DMA,
        ],
        compiler_params=pltpu.CompilerParams(needs_layout_passes=False),
    )(ids_t, table2, pos_flat)


def kernel(token_ids, token_table, pos_table):
    ids_t = token_ids.astype(jnp.int32).T
    tt = token_table.T
    out = _run(ids_t, tt, pos_table.reshape(-1))
    return jnp.transpose(out, (2, 0, 1))
